# baseline (speedup 1.0000x reference)
import functools

import jax
import jax.numpy as jnp
from jax import lax
from jax.experimental import pallas as pl
from jax.experimental.pallas import tpu as pltpu
from jax.experimental.pallas import tpu_sc as plsc

NC = 2
NS = 16
NW = NC * NS
CH = 128
K = 4
BLK = CH * K


def _gather_sc(table, idx2d):
    V, D = table.shape
    B = idx2d.shape[0] * CH
    b_per_w = B // NW
    n_blk = b_per_w // BLK
    rows_per_w = b_per_w // CH

    mesh = plsc.VectorSubcoreMesh(
        core_axis_name="c", subcore_axis_name="s", num_cores=NC, num_subcores=NS
    )

    @functools.partial(
        pl.kernel,
        out_type=jax.ShapeDtypeStruct((B, D), jnp.float32),
        mesh=mesh,
        compiler_params=pltpu.CompilerParams(use_tc_tiling_on_sc=False),
        scratch_types=[
            pltpu.VMEM((rows_per_w, CH), jnp.int32),
            pltpu.VMEM((BLK, D), jnp.float32),
            pltpu.SemaphoreType.DMA,
        ],
    )
    def grab(table_hbm, idx_hbm, out_hbm, idx_v, buf, sem):
        wid = lax.axis_index("s") * NC + lax.axis_index("c")
        base = wid * b_per_w
        pltpu.sync_copy(idx_hbm.at[pl.ds(wid * rows_per_w, rows_per_w)], idx_v)

        def block(g, _):
            for b in range(K):
                pltpu.async_copy(
                    table_hbm.at[idx_v.at[g * K + b]],
                    buf.at[pl.ds(b * CH, CH)],
                    sem,
                )
            for b in range(K):
                pltpu.make_async_copy(
                    table_hbm.at[idx_v.at[g * K + b]],
                    buf.at[pl.ds(b * CH, CH)],
                    sem,
                ).wait()
            pltpu.sync_copy(buf, out_hbm.at[pl.ds(base + g * BLK, BLK)])
            return ()

        lax.fori_loop(0, n_blk, block, (), unroll=False)

    return grab(table, idx2d)


def _project_tc(e, w):
    B, D = e.shape
    KV = w.shape[0]
    R = 4096

    def body(x_ref, w_ref, o_ref):
        o_ref[...] = lax.dot_general(
            x_ref[...], w_ref[...],
            dimension_numbers=(((1,), (1,)), ((), ())),
            preferred_element_type=jnp.float32,
        )

    return pl.pallas_call(
        body,
        grid=(B // R,),
        in_specs=[
            pl.BlockSpec((R, D), lambda i: (i, 0)),
            pl.BlockSpec((KV, D), lambda i: (0, 0)),
        ],
        out_specs=pl.BlockSpec((R, KV), lambda i: (i, 0)),
        out_shape=jax.ShapeDtypeStruct((B, KV), jnp.float32),
    )(e, w)


def kernel(input_ids, embed_weight, proj_weight):
    Bb, Ll = input_ids.shape
    idx2d = input_ids.reshape(-1, CH).astype(jnp.int32)
    e = _gather_sc(embed_weight, idx2d)
    out = _project_tc(e, proj_weight)
    return out.reshape(Bb, Ll, proj_weight.shape[0])


# speedup vs baseline: 1.2043x; 1.2043x over previous
import functools

import jax
import jax.numpy as jnp
from jax import lax
from jax.experimental import pallas as pl
from jax.experimental.pallas import tpu as pltpu
from jax.experimental.pallas import tpu_sc as plsc

NC = 2
NS = 16
NW = NC * NS
CH = 128
K = 4
BLK = CH * K


def _gather_sc(table, idx2d):
    V, D = table.shape
    B = idx2d.shape[0] * CH
    b_per_w = B // NW
    n_blk = b_per_w // BLK
    rows_per_w = b_per_w // CH

    mesh = plsc.VectorSubcoreMesh(
        core_axis_name="c", subcore_axis_name="s", num_cores=NC, num_subcores=NS
    )

    @functools.partial(
        pl.kernel,
        out_type=jax.ShapeDtypeStruct((B, D), jnp.float32),
        mesh=mesh,
        compiler_params=pltpu.CompilerParams(use_tc_tiling_on_sc=False),
        scratch_types=[
            pltpu.VMEM((rows_per_w, CH), jnp.int32),
            pltpu.VMEM((BLK, D), jnp.float32),
            pltpu.SemaphoreType.DMA,
        ],
    )
    def grab(table_hbm, idx_hbm, out_hbm, idx_v, buf, sem):
        wid = lax.axis_index("s") * NC + lax.axis_index("c")
        base = wid * b_per_w
        pltpu.sync_copy(idx_hbm.at[pl.ds(wid * rows_per_w, rows_per_w)], idx_v)

        def block(g, _):
            for b in range(K):
                pltpu.async_copy(
                    table_hbm.at[idx_v.at[g * K + b]],
                    buf.at[pl.ds(b * CH, CH)],
                    sem,
                )
            for b in range(K):
                pltpu.make_async_copy(
                    table_hbm.at[idx_v.at[g * K + b]],
                    buf.at[pl.ds(b * CH, CH)],
                    sem,
                ).wait()
            pltpu.sync_copy(buf, out_hbm.at[pl.ds(base + g * BLK, BLK)])
            return ()

        lax.fori_loop(0, n_blk, block, (), unroll=False)

    return grab(table, idx2d)


def _project_tc(e2, w, Bb, Ll):
    B2, D2 = e2.shape
    D = D2 // 2
    KV = w.shape[0]
    GB = 32
    rows2 = GB * Ll // 2

    def body(x_ref, w_ref, o_ref):
        x = x_ref[...]
        wt = w_ref[...]
        ya = lax.dot_general(
            x[:, :D], wt, dimension_numbers=(((1,), (1,)), ((), ())),
            preferred_element_type=jnp.float32,
        )
        yb = lax.dot_general(
            x[:, D:], wt, dimension_numbers=(((1,), (1,)), ((), ())),
            preferred_element_type=jnp.float32,
        )
        pair = jnp.concatenate([ya[:, None, :], yb[:, None, :]], axis=1)
        o_ref[...] = pair.reshape(GB, Ll, KV)

    return pl.pallas_call(
        body,
        grid=(Bb // GB,),
        in_specs=[
            pl.BlockSpec((rows2, D2), lambda i: (i, 0)),
            pl.BlockSpec((KV, D), lambda i: (0, 0)),
        ],
        out_specs=pl.BlockSpec((GB, Ll, KV), lambda i: (i, 0, 0)),
        out_shape=jax.ShapeDtypeStruct((Bb, Ll, KV), jnp.float32),
    )(e2, w)


def kernel(input_ids, embed_weight, proj_weight):
    Bb, Ll = input_ids.shape
    idx2d = input_ids.reshape(-1, CH).astype(jnp.int32)
    e = _gather_sc(embed_weight, idx2d)
    e2 = e.reshape(e.shape[0] // 2, 2 * e.shape[1])
    return _project_tc(e2, proj_weight, Bb, Ll)


# speedup vs baseline: 1.4230x; 1.1816x over previous
import functools

import jax
import jax.numpy as jnp
from jax import lax
from jax.experimental import pallas as pl
from jax.experimental.pallas import tpu as pltpu
from jax.experimental.pallas import tpu_sc as plsc

NC = 2
NS = 16
NW = NC * NS
CH = 128
K = 4
BLK = CH * K


def _gather_sc(table, idx2d):
    V, D = table.shape
    B = idx2d.shape[0] * CH
    b_per_w = B // NW
    n_blk = b_per_w // BLK
    rows_per_w = b_per_w // CH

    mesh = plsc.VectorSubcoreMesh(
        core_axis_name="c", subcore_axis_name="s", num_cores=NC, num_subcores=NS
    )

    @functools.partial(
        pl.kernel,
        out_type=jax.ShapeDtypeStruct((B, D), jnp.float32),
        mesh=mesh,
        compiler_params=pltpu.CompilerParams(use_tc_tiling_on_sc=False),
        scratch_types=[
            pltpu.VMEM((rows_per_w, CH), jnp.int32),
            pltpu.VMEM((BLK, D), jnp.float32),
            pltpu.SemaphoreType.DMA,
        ],
    )
    def grab(table_hbm, idx_hbm, out_hbm, idx_v, buf, sem):
        wid = lax.axis_index("s") * NC + lax.axis_index("c")
        base = wid * b_per_w
        pltpu.sync_copy(idx_hbm.at[pl.ds(wid * rows_per_w, rows_per_w)], idx_v)

        def block(g, _):
            for b in range(K):
                pltpu.async_copy(
                    table_hbm.at[idx_v.at[g * K + b]],
                    buf.at[pl.ds(b * CH, CH)],
                    sem,
                )
            for b in range(K):
                pltpu.make_async_copy(
                    table_hbm.at[idx_v.at[g * K + b]],
                    buf.at[pl.ds(b * CH, CH)],
                    sem,
                ).wait()
            pltpu.sync_copy(buf, out_hbm.at[pl.ds(base + g * BLK, BLK)])
            return ()

        lax.fori_loop(0, n_blk, block, (), unroll=False)

    return grab(table, idx2d)


def _project_tc(e2, wbig, Bb, Ll):
    B2, D2 = e2.shape
    KV2 = wbig.shape[1]
    KV = KV2 // 2
    GB = 64
    rows2 = GB * Ll // 2

    def body(x_ref, w_ref, o_ref):
        y = lax.dot_general(
            x_ref[...], w_ref[...],
            dimension_numbers=(((1,), (0,)), ((), ())),
            preferred_element_type=jnp.float32,
        )
        o_ref[...] = y.reshape(rows2, 2, KV).reshape(2 * rows2, KV).reshape(
            GB, Ll, KV
        )

    return pl.pallas_call(
        body,
        grid=(Bb // GB,),
        in_specs=[
            pl.BlockSpec((rows2, D2), lambda i: (i, 0)),
            pl.BlockSpec((D2, KV2), lambda i: (0, 0)),
        ],
        out_specs=pl.BlockSpec((GB, Ll, KV), lambda i: (i, 0, 0)),
        out_shape=jax.ShapeDtypeStruct((Bb, Ll, KV), jnp.float32),
    )(e2, wbig)


def kernel(input_ids, embed_weight, proj_weight):
    Bb, Ll = input_ids.shape
    KV, D = proj_weight.shape
    idx2d = input_ids.reshape(-1, CH).astype(jnp.int32)
    e = _gather_sc(embed_weight, idx2d)
    e2 = e.reshape(e.shape[0] // 2, 2 * e.shape[1])
    wt = proj_weight.T
    zero = jnp.zeros((D, KV), jnp.float32)
    wbig = jnp.block([[wt, zero], [zero, wt]])
    return _project_tc(e2, wbig, Bb, Ll)


# speedup vs baseline: 2.1309x; 1.4975x over previous
import functools

import jax
import jax.numpy as jnp
from jax import lax
from jax.experimental import pallas as pl
from jax.experimental.pallas import tpu as pltpu
from jax.experimental.pallas import tpu_sc as plsc

NC = 2
NS = 16
NW = NC * NS
CH = 128
K = 4
BLK = CH * K
PC = 8192
PH = PC // 2


def _pack_table_tc(tableT):
    D, V = tableT.shape
    NB = (V + PC - 1) // PC
    eye = jnp.eye(D, dtype=jnp.float32)

    def body(x_ref, i_ref, o_ref):
        x = x_ref[...]
        ident = i_ref[...]
        ya = lax.dot_general(
            x[:, :PH], ident, dimension_numbers=(((0,), (0,)), ((), ())),
            preferred_element_type=jnp.float32,
        )
        yb = lax.dot_general(
            x[:, PH:], ident, dimension_numbers=(((0,), (0,)), ((), ())),
            preferred_element_type=jnp.float32,
        )
        o_ref[...] = jnp.concatenate([ya, yb], axis=1)

    return pl.pallas_call(
        body,
        grid=(NB,),
        in_specs=[
            pl.BlockSpec((D, PC), lambda i: (0, i)),
            pl.BlockSpec((D, D), lambda i: (0, 0)),
        ],
        out_specs=pl.BlockSpec((PH, 2 * D), lambda i: (i, 0)),
        out_shape=jax.ShapeDtypeStruct((NB * PH, 2 * D), jnp.float32),
    )(tableT, eye)


def _gather_sc(table, idx2d):
    VP, D = table.shape
    B = idx2d.shape[0] * CH
    b_per_w = B // NW
    n_blk = b_per_w // BLK
    rows_per_w = b_per_w // CH

    mesh = plsc.VectorSubcoreMesh(
        core_axis_name="c", subcore_axis_name="s", num_cores=NC, num_subcores=NS
    )

    @functools.partial(
        pl.kernel,
        out_type=jax.ShapeDtypeStruct((B, D), jnp.float32),
        mesh=mesh,
        compiler_params=pltpu.CompilerParams(use_tc_tiling_on_sc=False),
        scratch_types=[
            pltpu.VMEM((rows_per_w, CH), jnp.int32),
            pltpu.VMEM((BLK, D), jnp.float32),
            pltpu.SemaphoreType.DMA,
        ],
    )
    def grab(table_hbm, idx_hbm, out_hbm, idx_v, buf, sem):
        wid = lax.axis_index("s") * NC + lax.axis_index("c")
        base = wid * b_per_w
        pltpu.sync_copy(idx_hbm.at[pl.ds(wid * rows_per_w, rows_per_w)], idx_v)

        def remap(j, _):
            def remap16(c, _):
                r = idx_v[j, pl.ds(c * 16, 16)]
                p = lax.bitwise_and(r, PC - 1)
                gbase = lax.sub(r, p)
                off = lax.select(
                    p < PH, lax.shift_left(p, 1), 2 * p - (PC - 1)
                )
                idx_v[j, pl.ds(c * 16, 16)] = lax.add(gbase, off)
                return ()

            lax.fori_loop(0, CH // 16, remap16, (), unroll=True)
            return ()

        lax.fori_loop(0, rows_per_w, remap, (), unroll=False)

        def block(g, _):
            for b in range(K):
                pltpu.async_copy(
                    table_hbm.at[idx_v.at[g * K + b]],
                    buf.at[pl.ds(b * CH, CH)],
                    sem,
                )
            for b in range(K):
                pltpu.make_async_copy(
                    table_hbm.at[idx_v.at[g * K + b]],
                    buf.at[pl.ds(b * CH, CH)],
                    sem,
                ).wait()
            pltpu.sync_copy(buf, out_hbm.at[pl.ds(base + g * BLK, BLK)])
            return ()

        lax.fori_loop(0, n_blk, block, (), unroll=False)

    return grab(table, idx2d)


def _project_tc(e2, wbig, Bb, Ll):
    B2, D2 = e2.shape
    KV2 = wbig.shape[1]
    KV = KV2 // 2
    GB = 64
    rows2 = GB * Ll // 2

    def body(x_ref, w_ref, o_ref):
        y = lax.dot_general(
            x_ref[...], w_ref[...],
            dimension_numbers=(((1,), (0,)), ((), ())),
            preferred_element_type=jnp.float32,
        )
        o_ref[...] = y.reshape(rows2, 2, KV).reshape(2 * rows2, KV).reshape(
            GB, Ll, KV
        )

    return pl.pallas_call(
        body,
        grid=(Bb // GB,),
        in_specs=[
            pl.BlockSpec((rows2, D2), lambda i: (i, 0)),
            pl.BlockSpec((D2, KV2), lambda i: (0, 0)),
        ],
        out_specs=pl.BlockSpec((GB, Ll, KV), lambda i: (i, 0, 0)),
        out_shape=jax.ShapeDtypeStruct((Bb, Ll, KV), jnp.float32),
    )(e2, wbig)


def kernel(input_ids, embed_weight, proj_weight):
    Bb, Ll = input_ids.shape
    KV, D = proj_weight.shape
    idx2d = input_ids.reshape(-1, CH).astype(jnp.int32)
    tableP = _pack_table_tc(embed_weight.T)
    table_lin = tableP.reshape(tableP.shape[0] * 2, D)
    e = _gather_sc(table_lin, idx2d)
    e2 = e.reshape(e.shape[0] // 2, 2 * e.shape[1])
    wt = proj_weight.T
    zero = jnp.zeros((D, KV), jnp.float32)
    wbig = jnp.block([[wt, zero], [zero, wt]])
    return _project_tc(e2, wbig, Bb, Ll)
